# Initial kernel scaffold; baseline (speedup 1.0000x reference)
import jax
import jax.numpy as jnp
from jax.experimental import pallas as pl


def kernel(x, edge_index, Wenc, benc, W1, b1, g_mid, bt_mid, W2, b2, eps, g_out, bt_out, Wout, bout):
    raise NotImplementedError("write your pallas kernel here")


# speedup vs baseline: 4.4956x; 4.4956x over previous
import functools

import jax
import jax.numpy as jnp
from jax import lax
from jax.experimental import pallas as pl
from jax.experimental.pallas import tpu as pltpu
from jax.experimental.pallas import tpu_sc as plsc

_N = 10000
_E = 320000
_D = 128
_H = 128
_L = 3

_NC = 2
_NS = 16
_NW = _NC * _NS
_EPW = _E // _NW
_CHUNK = 80
_STEPS = _EPW // _CHUNK
_NP = 10240
_RPS = _NP // _NS
_ZROWS = 128


def _sc_partials_body(h_hbm, src_hbm, dst_hbm, out_hbm,
                      sidx, didx, rows, zv, acc, sem):
    cid = lax.axis_index("c")
    sid = lax.axis_index("s")
    wid = sid * _NC + cid
    row0 = sid * _RPS

    @pl.loop(0, _ZROWS)
    def _zr(r):
        @pl.loop(0, _H, step=16)
        def _zc(c):
            zv[r, pl.ds(c, 16)] = jnp.zeros((16,), jnp.float32)

    @pl.loop(0, _RPS, step=_ZROWS)
    def _za(r):
        pltpu.sync_copy(zv, acc.at[pl.ds(row0 + r, _ZROWS)])

    plsc.subcore_barrier()

    base = wid * _EPW

    @pl.loop(0, _STEPS)
    def _main(s):
        off = base + s * _CHUNK
        pltpu.sync_copy(src_hbm.at[pl.ds(off, _CHUNK)], sidx)
        pltpu.async_copy(h_hbm.at[sidx], rows, sem).wait()
        pltpu.sync_copy(dst_hbm.at[pl.ds(off, _CHUNK)], didx)
        pltpu.sync_copy(rows, acc.at[didx], add=True)

    plsc.subcore_barrier()

    pltpu.sync_copy(acc.at[pl.ds(row0, _RPS)],
                    out_hbm.at[cid, pl.ds(row0, _RPS)])


@jax.jit
def _sc_partials(h, src, dst):
    kern = pl.kernel(
        _sc_partials_body,
        out_type=jax.ShapeDtypeStruct((_NC, _NP, _H), jnp.float32),
        mesh=plsc.VectorSubcoreMesh(core_axis_name="c", subcore_axis_name="s"),
        scratch_types=[
            pltpu.VMEM((_CHUNK,), jnp.int32),
            pltpu.VMEM((_CHUNK,), jnp.int32),
            pltpu.VMEM((_CHUNK, _H), jnp.float32),
            pltpu.VMEM((_ZROWS, _H), jnp.float32),
            pltpu.VMEM_SHARED((_NP, _H), jnp.float32),
            pltpu.SemaphoreType.DMA,
        ],
    )
    return kern(h, src, dst)


def _enc_body(x_ref, w_ref, b_ref, o_ref):
    o_ref[...] = (
        jnp.dot(x_ref[...], w_ref[...], preferred_element_type=jnp.float32)
        + b_ref[...]
    )


@jax.jit
def _encode(x, Wenc, benc):
    return pl.pallas_call(
        _enc_body,
        out_shape=jax.ShapeDtypeStruct((_N, _H), jnp.float32),
    )(x, Wenc, benc.reshape(1, _H))


def _bn_relu(z, g, b):
    m = jnp.mean(z, axis=0, keepdims=True)
    v = jnp.mean(jnp.square(z - m), axis=0, keepdims=True)
    z = g * (z - m) / jnp.sqrt(v + 1e-5) + b
    return jnp.maximum(z, 0.0)


def _gin_mlp(h_ref, p_ref, w1_ref, b1_ref, gm_ref, bm_ref,
             w2_ref, b2_ref, go_ref, bo_ref, sc_ref):
    h = h_ref[...]
    z = sc_ref[...] * h + (p_ref[0, : _N] + p_ref[1, : _N])
    z = jnp.dot(z, w1_ref[...], preferred_element_type=jnp.float32) + b1_ref[...]
    z = _bn_relu(z, gm_ref[...], bm_ref[...])
    z = jnp.dot(z, w2_ref[...], preferred_element_type=jnp.float32) + b2_ref[...]
    z = _bn_relu(z, go_ref[...], bo_ref[...])
    return z + h


def _layer_body(h_ref, p_ref, w1_ref, b1_ref, gm_ref, bm_ref,
                w2_ref, b2_ref, go_ref, bo_ref, sc_ref, o_ref):
    o_ref[...] = _gin_mlp(h_ref, p_ref, w1_ref, b1_ref, gm_ref, bm_ref,
                          w2_ref, b2_ref, go_ref, bo_ref, sc_ref)


def _last_body(h_ref, p_ref, w1_ref, b1_ref, gm_ref, bm_ref,
               w2_ref, b2_ref, go_ref, bo_ref, sc_ref,
               wo_ref, bo2_ref, o_ref):
    hn = _gin_mlp(h_ref, p_ref, w1_ref, b1_ref, gm_ref, bm_ref,
                  w2_ref, b2_ref, go_ref, bo_ref, sc_ref)
    o_ref[...] = (
        jnp.dot(hn, wo_ref[...], preferred_element_type=jnp.float32)
        + bo2_ref[...]
    )


@jax.jit
def _layer(*args):
    return pl.pallas_call(
        _layer_body,
        out_shape=jax.ShapeDtypeStruct((_N, _H), jnp.float32),
    )(*args)


@jax.jit
def _last(*args):
    return pl.pallas_call(
        _last_body,
        out_shape=jax.ShapeDtypeStruct((_N, 1), jnp.float32),
    )(*args)


def kernel(x, edge_index, Wenc, benc, W1, b1, g_mid, bt_mid, W2, b2,
           eps, g_out, bt_out, Wout, bout):
    src = edge_index[0].astype(jnp.int32)
    dst = edge_index[1].astype(jnp.int32)
    h = _encode(x, Wenc, benc)
    for i in range(_L):
        parts = _sc_partials(h, src, dst)
        sc = (1.0 + eps[i]) * jnp.ones((1, _H), jnp.float32)
        args = (h, parts, W1[i], b1[i].reshape(1, -1),
                g_mid[i].reshape(1, -1), bt_mid[i].reshape(1, -1),
                W2[i], b2[i].reshape(1, -1),
                g_out[i].reshape(1, -1), bt_out[i].reshape(1, -1), sc)
        if i < _L - 1:
            h = _layer(*args)
        else:
            out = _last(*args, Wout, bout.reshape(1, 1))
    return out
